# baseline (speedup 1.0000x reference)
import functools

import jax
import jax.numpy as jnp
from jax import lax
from jax.experimental import pallas as pl
from jax.experimental.pallas import tpu as pltpu
from jax.experimental.pallas import tpu_sc as plsc

B, MAXN, F, H, T, E, L = 16, 625, 128, 128, 8, 320000, 2
N = B * MAXN
PHID = 64

NC, NS = 2, 16
NW = NC * NS
GLEN = 64
RD = 4
GTOT = 327680 // GLEN
GP0 = GTOT // NS
NPARTS = 1
SCH = 16
EPAD = GTOT * GLEN
NROWS_PER_TILE = 632
NP = NS * NROWS_PER_TILE

_f32 = jnp.float32


def _hin_body(nf_ref, w_ref, b_ref, out_ref):
    out_ref[...] = jnp.tanh(
        jnp.dot(nf_ref[...], w_ref[...], preferred_element_type=_f32)
        + b_ref[...])


def _hin(nf, w, b):
    blk = 1000
    return pl.pallas_call(
        _hin_body,
        grid=(N // blk,),
        in_specs=[
            pl.BlockSpec((blk, F), lambda i: (i, 0)),
            pl.BlockSpec((F, H), lambda i: (0, 0)),
            pl.BlockSpec((1, H), lambda i: (0, 0)),
        ],
        out_specs=pl.BlockSpec((blk, H), lambda i: (i, 0)),
        out_shape=jax.ShapeDtypeStruct((N, H), _f32),
    )(nf, w, b)


def _expand_body(h_ref, w_ref, b_ref, out_ref):
    out_ref[...] = (
        jnp.dot(h_ref[...], w_ref[...], preferred_element_type=_f32)
        + b_ref[0])


def _expand(h, w_flat, b):
    blk = 1000
    nb = N // blk
    return pl.pallas_call(
        _expand_body,
        grid=(nb, T),
        in_specs=[
            pl.BlockSpec((blk, H), lambda i, t: (i, 0)),
            pl.BlockSpec((H, H), lambda i, t: (t, 0)),
            pl.BlockSpec((1, 1, H), lambda i, t: (t, 0, 0)),
        ],
        out_specs=pl.BlockSpec((blk, H), lambda i, t: (t * nb + i, 0)),
        out_shape=jax.ShapeDtypeStruct((T * N, H), _f32),
    )(h, w_flat, b)


def _edge_agg_body(table, gidx, dst, zeros, out, idx_v, dst_v, *rest):
    bufs = rest[:RD]
    sems = rest[RD:2 * RD]
    agg_sh = rest[2 * RD]
    c = lax.axis_index("c")
    s = lax.axis_index("s")
    base_g = s * GP0
    nchunks = jnp.where(c == 0, GP0 // SCH, 0)
    @pl.when(c == 0)
    def _():
        pltpu.sync_copy(zeros.at[pl.ds(s * NROWS_PER_TILE, NROWS_PER_TILE)],
                        agg_sh.at[pl.ds(s * NROWS_PER_TILE, NROWS_PER_TILE)])
    plsc.subcore_barrier()

    def chunk_body(ci, carry):
        pltpu.sync_copy(gidx.at[pl.ds(base_g + ci * SCH, SCH)], idx_v)
        pltpu.sync_copy(dst.at[pl.ds(base_g + ci * SCH, SCH)], dst_v)
        for b in range(RD):
            pltpu.async_copy(table.at[idx_v.at[b]], bufs[b], sems[b])

        def body(g, carry2):
            for b in range(RD):
                j = RD * g + b
                pltpu.make_async_copy(
                    table.at[idx_v.at[j]], bufs[b], sems[b]).wait()
                pltpu.sync_copy(bufs[b], agg_sh.at[dst_v.at[j]], add=True)

                @pl.when(j + RD < SCH)
                def _():
                    pltpu.async_copy(
                        table.at[idx_v.at[j + RD]], bufs[b], sems[b])
            return carry2

        lax.fori_loop(0, SCH // RD, body, 0)
        return carry

    lax.fori_loop(0, nchunks, chunk_body, 0)
    plsc.subcore_barrier()

    @pl.when(c == 0)
    def _():
        pltpu.sync_copy(agg_sh.at[pl.ds(s * NROWS_PER_TILE, NROWS_PER_TILE)],
                        out.at[0, pl.ds(s * NROWS_PER_TILE, NROWS_PER_TILE)])


_edge_agg = functools.partial(
    pl.kernel,
    out_type=jax.ShapeDtypeStruct((NPARTS, NP, H), _f32),
    mesh=plsc.VectorSubcoreMesh(
        core_axis_name="c", subcore_axis_name="s",
        num_cores=NC, num_subcores=NS),
    scratch_types=[
        pltpu.VMEM((SCH, GLEN), jnp.int32),
        pltpu.VMEM((SCH, GLEN), jnp.int32),
    ]
    + [pltpu.VMEM((GLEN, H), _f32) for _ in range(RD)]
    + [pltpu.SemaphoreType.DMA for _ in range(RD)]
    + [pltpu.VMEM_SHARED((NP, H), _f32)],
)(_edge_agg_body)


def _gru_body(*refs):
    (h_ref, wz_ref, uz_ref, bz_ref, wr_ref, ur_ref,
     br_ref, wn_ref, un_ref, bn_ref, out_ref) = refs[NPARTS:]
    agg = refs[0][...]
    for p in refs[1:NPARTS]:
        agg = agg + p[...]
    h = h_ref[...]
    dot = lambda a, b: jnp.dot(a, b[...], preferred_element_type=_f32)
    z = jax.nn.sigmoid(dot(agg, wz_ref) + dot(h, uz_ref) + bz_ref[...])
    r = jax.nn.sigmoid(dot(agg, wr_ref) + dot(h, ur_ref) + br_ref[...])
    cand = jnp.tanh(dot(agg, wn_ref) + r * dot(h, un_ref) + bn_ref[...])
    out_ref[...] = (1.0 - z) * cand + z * h


def _gru(parts, h, wz, uz, bz, wr, ur, br, wn, un, bn):
    blk = 1000
    row = pl.BlockSpec((blk, H), lambda i: (i, 0))
    mat = pl.BlockSpec((H, H), lambda i: (0, 0))
    vec = pl.BlockSpec((1, H), lambda i: (0, 0))
    return pl.pallas_call(
        _gru_body,
        grid=(N // blk,),
        in_specs=[row] * NPARTS
        + [row, mat, mat, vec, mat, mat, vec, mat, mat, vec],
        out_specs=row,
        out_shape=jax.ShapeDtypeStruct((N, H), _f32),
    )(*parts, h, wz, uz, bz, wr, ur, br, wn, un, bn)


def _head_body(hb_ref, nn_ref, inh_ref, wp1a_ref, wp1b_ref, bp1_ref,
               wp2_ref, bp2_ref, wprob_ref, bprob_ref, wconf_ref, bconf_ref,
               wc1_ref, bc1_ref, wc2_ref, bc2_ref,
               logits_ref, prob_ref, conf_ref, upper_ref, lower_ref):
    hb = hb_ref[...]
    nn = nn_ref[...]
    iota = lax.broadcasted_iota(jnp.int32, (B, MAXN), 1)
    mask = (iota < nn).astype(_f32)
    denom = jnp.maximum(jnp.sum(mask, axis=1, keepdims=True), 1.0)
    lower = jnp.sum(hb * mask[:, :, None], axis=1) / denom
    dot = lambda a, b: jnp.dot(a, b[...], preferred_element_type=_f32)
    hp = jax.nn.relu(dot(inh_ref[...], wp1a_ref) + dot(lower, wp1b_ref)
                     + bp1_ref[...])
    hp = jax.nn.relu(dot(hp, wp2_ref) + bp2_ref[...])
    prob = jax.nn.sigmoid(dot(hp, wprob_ref) + bprob_ref[...])
    conf = jax.nn.sigmoid(dot(hp, wconf_ref) + bconf_ref[...])
    hc = jax.nn.relu(dot(lower, wc1_ref) + bc1_ref[...])
    logits_ref[...] = jax.nn.sigmoid(dot(hc, wc2_ref) + bc2_ref[...])
    prob_ref[...] = prob
    conf_ref[...] = conf
    upper_ref[...] = (prob >= 0.5).astype(_f32)
    lower_ref[...] = lower


def _heads(hb, nn, inh, wp1a, wp1b, bp1, wp2, bp2, wprob, bprob, wconf, bconf,
           wc1, bc1, wc2, bc2):
    args = (hb, nn, inh, wp1a, wp1b, bp1, wp2, bp2, wprob, bprob, wconf,
            bconf, wc1, bc1, wc2, bc2)
    return pl.pallas_call(
        _head_body,
        out_shape=[
            jax.ShapeDtypeStruct((B, 1), _f32),
            jax.ShapeDtypeStruct((B, 1), _f32),
            jax.ShapeDtypeStruct((B, 1), _f32),
            jax.ShapeDtypeStruct((B, 1), _f32),
            jax.ShapeDtypeStruct((B, H), _f32),
        ],
    )(*args)


def kernel(node_features, edge_index, edge_type, num_nodes,
           inheritance_features, W_in, b_in, W_msg, b_msg, Wz, Uz, bz,
           Wr, Ur, br, Wn, Un, bn, Wp1, bp1, Wp2, bp2, Wprob, bprob,
           Wconf, bconf, Wc1, bc1, Wc2, bc2):
    nf = node_features.reshape(N, F)
    h = _hin(nf, W_in, b_in.reshape(1, H))

    src = edge_index[0]
    dst = edge_index[1]
    pad = EPAD - E
    gidx = (edge_type * N + src).astype(jnp.int32)
    gidx_p = jnp.concatenate(
        [gidx, jnp.zeros((pad,), jnp.int32)]).reshape(EPAD // GLEN, GLEN)
    dst_p = jnp.concatenate(
        [dst.astype(jnp.int32),
         jnp.full((pad,), N, jnp.int32)]).reshape(EPAD // GLEN, GLEN)
    zeros = jnp.zeros((NP, H), _f32)

    for l in range(L):
        table = _expand(h, W_msg[l].reshape(T * H, H),
                        b_msg[l].reshape(T, 1, H))
        parts = _edge_agg(table, gidx_p, dst_p, zeros)
        h = _gru([parts[i, :N] for i in range(NPARTS)], h,
                 Wz[l], Uz[l], bz[l].reshape(1, H),
                 Wr[l], Ur[l], br[l].reshape(1, H),
                 Wn[l], Un[l], bn[l].reshape(1, H))

    logits, prob, conf, upper, lower = _heads(
        h.reshape(B, MAXN, H), num_nodes.reshape(B, 1),
        inheritance_features,
        Wp1[:8], Wp1[8:], bp1.reshape(1, PHID),
        Wp2, bp2.reshape(1, PHID),
        Wprob, bprob.reshape(1, 1),
        Wconf, bconf.reshape(1, 1),
        Wc1, bc1.reshape(1, H // 2),
        Wc2, bc2.reshape(1, 1))
    return logits, prob, conf, upper, lower


# speedup vs baseline: 1.2559x; 1.2559x over previous
import functools

import jax
import jax.numpy as jnp
from jax import lax
from jax.experimental import pallas as pl
from jax.experimental.pallas import tpu as pltpu
from jax.experimental.pallas import tpu_sc as plsc

B, MAXN, F, H, T, E, L = 16, 625, 128, 128, 8, 320000, 2
N = B * MAXN
PHID = 64

NC, NS = 2, 16
NW = NC * NS
GLEN = 64
RD = 4
GTOT = 327680 // GLEN
GP0 = 288
GP1 = 32
SCH = 16
EPAD = GTOT * GLEN
NROWS_PER_TILE = 632
NP = NS * NROWS_PER_TILE

_f32 = jnp.float32


def _hin_body(nf_ref, w_ref, b_ref, out_ref):
    out_ref[...] = jnp.tanh(
        jnp.dot(nf_ref[...], w_ref[...], preferred_element_type=_f32)
        + b_ref[...])


def _hin(nf, w, b):
    blk = 1000
    return pl.pallas_call(
        _hin_body,
        grid=(N // blk,),
        in_specs=[
            pl.BlockSpec((blk, F), lambda i: (i, 0)),
            pl.BlockSpec((F, H), lambda i: (0, 0)),
            pl.BlockSpec((1, H), lambda i: (0, 0)),
        ],
        out_specs=pl.BlockSpec((blk, H), lambda i: (i, 0)),
        out_shape=jax.ShapeDtypeStruct((N, H), _f32),
    )(nf, w, b)


def _expand_body(h_ref, w_ref, b_ref, out_ref):
    out_ref[...] = (
        jnp.dot(h_ref[...], w_ref[...], preferred_element_type=_f32)
        + b_ref[0])


def _expand(h, w_flat, b):
    blk = 1000
    nb = N // blk
    return pl.pallas_call(
        _expand_body,
        grid=(nb, T),
        in_specs=[
            pl.BlockSpec((blk, H), lambda i, t: (i, 0)),
            pl.BlockSpec((H, H), lambda i, t: (t, 0)),
            pl.BlockSpec((1, 1, H), lambda i, t: (t, 0, 0)),
        ],
        out_specs=pl.BlockSpec((blk, H), lambda i, t: (t * nb + i, 0)),
        out_shape=jax.ShapeDtypeStruct((T * N, H), _f32),
    )(h, w_flat, b)


def _edge_agg_body(table, gidx, dst, zeros, out, idx_v, dst_v, *rest):
    bufs = rest[:RD]
    sems = rest[RD:2 * RD]
    agg_sh = rest[2 * RD]
    c = lax.axis_index("c")
    s = lax.axis_index("s")
    base_g = jnp.where(c == 0, s * GP0, NS * GP0 + s * GP1)
    nchunks = jnp.where(c == 0, GP0 // SCH, GP1 // SCH)
    pltpu.sync_copy(zeros.at[pl.ds(s * NROWS_PER_TILE, NROWS_PER_TILE)],
                    agg_sh.at[pl.ds(s * NROWS_PER_TILE, NROWS_PER_TILE)])
    plsc.subcore_barrier()

    def chunk_body(ci, carry):
        pltpu.sync_copy(gidx.at[pl.ds(base_g + ci * SCH, SCH)], idx_v)
        pltpu.sync_copy(dst.at[pl.ds(base_g + ci * SCH, SCH)], dst_v)
        for b in range(RD):
            pltpu.async_copy(table.at[idx_v.at[b]], bufs[b], sems[b])

        def body(g, carry2):
            for b in range(RD):
                j = RD * g + b
                pltpu.make_async_copy(
                    table.at[idx_v.at[j]], bufs[b], sems[b]).wait()
                pltpu.sync_copy(bufs[b], agg_sh.at[dst_v.at[j]], add=True)

                @pl.when(j + RD < SCH)
                def _():
                    pltpu.async_copy(
                        table.at[idx_v.at[j + RD]], bufs[b], sems[b])
            return carry2

        lax.fori_loop(0, SCH // RD, body, 0)
        return carry

    lax.fori_loop(0, nchunks, chunk_body, 0)
    plsc.subcore_barrier()
    pltpu.sync_copy(agg_sh.at[pl.ds(s * NROWS_PER_TILE, NROWS_PER_TILE)],
                    out.at[c, pl.ds(s * NROWS_PER_TILE, NROWS_PER_TILE)])


_edge_agg = functools.partial(
    pl.kernel,
    out_type=jax.ShapeDtypeStruct((NC, NP, H), _f32),
    mesh=plsc.VectorSubcoreMesh(
        core_axis_name="c", subcore_axis_name="s",
        num_cores=NC, num_subcores=NS),
    scratch_types=[
        pltpu.VMEM((SCH, GLEN), jnp.int32),
        pltpu.VMEM((SCH, GLEN), jnp.int32),
    ]
    + [pltpu.VMEM((GLEN, H), _f32) for _ in range(RD)]
    + [pltpu.SemaphoreType.DMA for _ in range(RD)]
    + [pltpu.VMEM_SHARED((NP, H), _f32)],
)(_edge_agg_body)


def _gru_body(*refs):
    (h_ref, wz_ref, uz_ref, bz_ref, wr_ref, ur_ref,
     br_ref, wn_ref, un_ref, bn_ref, out_ref) = refs[NC:]
    agg = refs[0][...]
    for p in refs[1:NC]:
        agg = agg + p[...]
    h = h_ref[...]
    dot = lambda a, b: jnp.dot(a, b[...], preferred_element_type=_f32)
    z = jax.nn.sigmoid(dot(agg, wz_ref) + dot(h, uz_ref) + bz_ref[...])
    r = jax.nn.sigmoid(dot(agg, wr_ref) + dot(h, ur_ref) + br_ref[...])
    cand = jnp.tanh(dot(agg, wn_ref) + r * dot(h, un_ref) + bn_ref[...])
    out_ref[...] = (1.0 - z) * cand + z * h


def _gru(parts, h, wz, uz, bz, wr, ur, br, wn, un, bn):
    blk = 1000
    row = pl.BlockSpec((blk, H), lambda i: (i, 0))
    mat = pl.BlockSpec((H, H), lambda i: (0, 0))
    vec = pl.BlockSpec((1, H), lambda i: (0, 0))
    return pl.pallas_call(
        _gru_body,
        grid=(N // blk,),
        in_specs=[row] * NC
        + [row, mat, mat, vec, mat, mat, vec, mat, mat, vec],
        out_specs=row,
        out_shape=jax.ShapeDtypeStruct((N, H), _f32),
    )(*parts, h, wz, uz, bz, wr, ur, br, wn, un, bn)


def _head_body(hb_ref, nn_ref, inh_ref, wp1a_ref, wp1b_ref, bp1_ref,
               wp2_ref, bp2_ref, wprob_ref, bprob_ref, wconf_ref, bconf_ref,
               wc1_ref, bc1_ref, wc2_ref, bc2_ref,
               logits_ref, prob_ref, conf_ref, upper_ref, lower_ref):
    hb = hb_ref[...]
    nn = nn_ref[...]
    iota = lax.broadcasted_iota(jnp.int32, (B, MAXN), 1)
    mask = (iota < nn).astype(_f32)
    denom = jnp.maximum(jnp.sum(mask, axis=1, keepdims=True), 1.0)
    lower = jnp.sum(hb * mask[:, :, None], axis=1) / denom
    dot = lambda a, b: jnp.dot(a, b[...], preferred_element_type=_f32)
    hp = jax.nn.relu(dot(inh_ref[...], wp1a_ref) + dot(lower, wp1b_ref)
                     + bp1_ref[...])
    hp = jax.nn.relu(dot(hp, wp2_ref) + bp2_ref[...])
    prob = jax.nn.sigmoid(dot(hp, wprob_ref) + bprob_ref[...])
    conf = jax.nn.sigmoid(dot(hp, wconf_ref) + bconf_ref[...])
    hc = jax.nn.relu(dot(lower, wc1_ref) + bc1_ref[...])
    logits_ref[...] = jax.nn.sigmoid(dot(hc, wc2_ref) + bc2_ref[...])
    prob_ref[...] = prob
    conf_ref[...] = conf
    upper_ref[...] = (prob >= 0.5).astype(_f32)
    lower_ref[...] = lower


def _heads(hb, nn, inh, wp1a, wp1b, bp1, wp2, bp2, wprob, bprob, wconf, bconf,
           wc1, bc1, wc2, bc2):
    args = (hb, nn, inh, wp1a, wp1b, bp1, wp2, bp2, wprob, bprob, wconf,
            bconf, wc1, bc1, wc2, bc2)
    return pl.pallas_call(
        _head_body,
        out_shape=[
            jax.ShapeDtypeStruct((B, 1), _f32),
            jax.ShapeDtypeStruct((B, 1), _f32),
            jax.ShapeDtypeStruct((B, 1), _f32),
            jax.ShapeDtypeStruct((B, 1), _f32),
            jax.ShapeDtypeStruct((B, H), _f32),
        ],
    )(*args)


def kernel(node_features, edge_index, edge_type, num_nodes,
           inheritance_features, W_in, b_in, W_msg, b_msg, Wz, Uz, bz,
           Wr, Ur, br, Wn, Un, bn, Wp1, bp1, Wp2, bp2, Wprob, bprob,
           Wconf, bconf, Wc1, bc1, Wc2, bc2):
    nf = node_features.reshape(N, F)
    h = _hin(nf, W_in, b_in.reshape(1, H))

    src = edge_index[0]
    dst = edge_index[1]
    pad = EPAD - E
    gidx = (edge_type * N + src).astype(jnp.int32)
    gidx_p = jnp.concatenate(
        [gidx, jnp.zeros((pad,), jnp.int32)]).reshape(EPAD // GLEN, GLEN)
    dst_p = jnp.concatenate(
        [dst.astype(jnp.int32),
         jnp.full((pad,), N, jnp.int32)]).reshape(EPAD // GLEN, GLEN)
    zeros = jnp.zeros((NP, H), _f32)

    for l in range(L):
        table = _expand(h, W_msg[l].reshape(T * H, H),
                        b_msg[l].reshape(T, 1, H))
        parts = _edge_agg(table, gidx_p, dst_p, zeros)
        h = _gru([parts[i, :N] for i in range(NC)], h,
                 Wz[l], Uz[l], bz[l].reshape(1, H),
                 Wr[l], Ur[l], br[l].reshape(1, H),
                 Wn[l], Un[l], bn[l].reshape(1, H))

    logits, prob, conf, upper, lower = _heads(
        h.reshape(B, MAXN, H), num_nodes.reshape(B, 1),
        inheritance_features,
        Wp1[:8], Wp1[8:], bp1.reshape(1, PHID),
        Wp2, bp2.reshape(1, PHID),
        Wprob, bprob.reshape(1, 1),
        Wconf, bconf.reshape(1, 1),
        Wc1, bc1.reshape(1, H // 2),
        Wc2, bc2.reshape(1, 1))
    return logits, prob, conf, upper, lower


# speedup vs baseline: 1.2586x; 1.0022x over previous
import functools

import jax
import jax.numpy as jnp
from jax import lax
from jax.experimental import pallas as pl
from jax.experimental.pallas import tpu as pltpu
from jax.experimental.pallas import tpu_sc as plsc

B, MAXN, F, H, T, E, L = 16, 625, 128, 128, 8, 320000, 2
N = B * MAXN
PHID = 64

NC, NS = 2, 16
NW = NC * NS
GLEN = 64
RD = 4
GTOT = 327680 // GLEN
GP0 = 288
GP1 = 32
SCH = 16
EPAD = GTOT * GLEN
NROWS_PER_TILE = 632
NP = NS * NROWS_PER_TILE

_f32 = jnp.float32


def _hin_body(nf_ref, w_ref, b_ref, out_ref):
    out_ref[...] = jnp.tanh(
        jnp.dot(nf_ref[...], w_ref[...], preferred_element_type=_f32)
        + b_ref[...])


def _hin(nf, w, b):
    blk = 1000
    return pl.pallas_call(
        _hin_body,
        grid=(N // blk,),
        in_specs=[
            pl.BlockSpec((blk, F), lambda i: (i, 0)),
            pl.BlockSpec((F, H), lambda i: (0, 0)),
            pl.BlockSpec((1, H), lambda i: (0, 0)),
        ],
        out_specs=pl.BlockSpec((blk, H), lambda i: (i, 0)),
        out_shape=jax.ShapeDtypeStruct((N, H), _f32),
    )(nf, w, b)


def _expand_body(h_ref, w_ref, b_ref, out_ref):
    out_ref[...] = (
        jnp.dot(h_ref[...], w_ref[...], preferred_element_type=_f32)
        + b_ref[0])


def _expand(h, w_flat, b):
    blk = 1000
    nb = N // blk
    return pl.pallas_call(
        _expand_body,
        grid=(nb, T),
        in_specs=[
            pl.BlockSpec((blk, H), lambda i, t: (i, 0)),
            pl.BlockSpec((H, H), lambda i, t: (t, 0)),
            pl.BlockSpec((1, 1, H), lambda i, t: (t, 0, 0)),
        ],
        out_specs=pl.BlockSpec((blk, H), lambda i, t: (t * nb + i, 0)),
        out_shape=jax.ShapeDtypeStruct((T * N, H), _f32),
    )(h, w_flat, b)


def _edge_agg_body(table, gidx, dst, out, idx_v, dst_v, *rest):
    bufs = rest[:RD]
    sems = rest[RD:2 * RD]
    agg_sh = rest[2 * RD]
    c = lax.axis_index("c")
    s = lax.axis_index("s")
    base_g = jnp.where(c == 0, s * GP0, NS * GP0 + s * GP1)
    nchunks = jnp.where(c == 0, GP0 // SCH, GP1 // SCH)

    zv = jnp.zeros((16,), _f32)

    def zrow(i, carry):
        for k in range(H // 16):
            bufs[0][i, pl.ds(k * 16, 16)] = zv
        return carry

    lax.fori_loop(0, GLEN, zrow, 0)
    nrep = NROWS_PER_TILE // GLEN
    tail = NROWS_PER_TILE - nrep * GLEN

    def zcopy(k, carry):
        pltpu.sync_copy(
            bufs[0],
            agg_sh.at[pl.ds(s * NROWS_PER_TILE + k * GLEN, GLEN)])
        return carry

    lax.fori_loop(0, nrep, zcopy, 0)
    pltpu.sync_copy(
        bufs[0].at[pl.ds(0, tail)],
        agg_sh.at[pl.ds(s * NROWS_PER_TILE + nrep * GLEN, tail)])
    plsc.subcore_barrier()

    def chunk_body(ci, carry):
        pltpu.sync_copy(gidx.at[pl.ds(base_g + ci * SCH, SCH)], idx_v)
        pltpu.sync_copy(dst.at[pl.ds(base_g + ci * SCH, SCH)], dst_v)
        for b in range(RD):
            pltpu.async_copy(table.at[idx_v.at[b]], bufs[b], sems[b])

        def body(g, carry2):
            for b in range(RD):
                j = RD * g + b
                pltpu.make_async_copy(
                    table.at[idx_v.at[j]], bufs[b], sems[b]).wait()
                pltpu.sync_copy(bufs[b], agg_sh.at[dst_v.at[j]], add=True)

                @pl.when(j + RD < SCH)
                def _():
                    pltpu.async_copy(
                        table.at[idx_v.at[j + RD]], bufs[b], sems[b])
            return carry2

        lax.fori_loop(0, SCH // RD, body, 0)
        return carry

    lax.fori_loop(0, nchunks, chunk_body, 0)
    plsc.subcore_barrier()
    pltpu.sync_copy(agg_sh.at[pl.ds(s * NROWS_PER_TILE, NROWS_PER_TILE)],
                    out.at[c, pl.ds(s * NROWS_PER_TILE, NROWS_PER_TILE)])


_edge_agg = functools.partial(
    pl.kernel,
    out_type=jax.ShapeDtypeStruct((NC, NP, H), _f32),
    mesh=plsc.VectorSubcoreMesh(
        core_axis_name="c", subcore_axis_name="s",
        num_cores=NC, num_subcores=NS),
    scratch_types=[
        pltpu.VMEM((SCH, GLEN), jnp.int32),
        pltpu.VMEM((SCH, GLEN), jnp.int32),
    ]
    + [pltpu.VMEM((GLEN, H), _f32) for _ in range(RD)]
    + [pltpu.SemaphoreType.DMA for _ in range(RD)]
    + [pltpu.VMEM_SHARED((NP, H), _f32)],
)(_edge_agg_body)


def _gru_body(*refs):
    (h_ref, wz_ref, uz_ref, bz_ref, wr_ref, ur_ref,
     br_ref, wn_ref, un_ref, bn_ref, out_ref) = refs[NC:]
    agg = refs[0][...]
    for p in refs[1:NC]:
        agg = agg + p[...]
    h = h_ref[...]
    dot = lambda a, b: jnp.dot(a, b[...], preferred_element_type=_f32)
    z = jax.nn.sigmoid(dot(agg, wz_ref) + dot(h, uz_ref) + bz_ref[...])
    r = jax.nn.sigmoid(dot(agg, wr_ref) + dot(h, ur_ref) + br_ref[...])
    cand = jnp.tanh(dot(agg, wn_ref) + r * dot(h, un_ref) + bn_ref[...])
    out_ref[...] = (1.0 - z) * cand + z * h


def _gru(parts, h, wz, uz, bz, wr, ur, br, wn, un, bn):
    blk = 1000
    row = pl.BlockSpec((blk, H), lambda i: (i, 0))
    mat = pl.BlockSpec((H, H), lambda i: (0, 0))
    vec = pl.BlockSpec((1, H), lambda i: (0, 0))
    return pl.pallas_call(
        _gru_body,
        grid=(N // blk,),
        in_specs=[row] * NC
        + [row, mat, mat, vec, mat, mat, vec, mat, mat, vec],
        out_specs=row,
        out_shape=jax.ShapeDtypeStruct((N, H), _f32),
    )(*parts, h, wz, uz, bz, wr, ur, br, wn, un, bn)


def _head_body(hb_ref, nn_ref, inh_ref, wp1a_ref, wp1b_ref, bp1_ref,
               wp2_ref, bp2_ref, wprob_ref, bprob_ref, wconf_ref, bconf_ref,
               wc1_ref, bc1_ref, wc2_ref, bc2_ref,
               logits_ref, prob_ref, conf_ref, upper_ref, lower_ref):
    hb = hb_ref[...]
    nn = nn_ref[...]
    iota = lax.broadcasted_iota(jnp.int32, (B, MAXN), 1)
    mask = (iota < nn).astype(_f32)
    denom = jnp.maximum(jnp.sum(mask, axis=1, keepdims=True), 1.0)
    lower = jnp.sum(hb * mask[:, :, None], axis=1) / denom
    dot = lambda a, b: jnp.dot(a, b[...], preferred_element_type=_f32)
    hp = jax.nn.relu(dot(inh_ref[...], wp1a_ref) + dot(lower, wp1b_ref)
                     + bp1_ref[...])
    hp = jax.nn.relu(dot(hp, wp2_ref) + bp2_ref[...])
    prob = jax.nn.sigmoid(dot(hp, wprob_ref) + bprob_ref[...])
    conf = jax.nn.sigmoid(dot(hp, wconf_ref) + bconf_ref[...])
    hc = jax.nn.relu(dot(lower, wc1_ref) + bc1_ref[...])
    logits_ref[...] = jax.nn.sigmoid(dot(hc, wc2_ref) + bc2_ref[...])
    prob_ref[...] = prob
    conf_ref[...] = conf
    upper_ref[...] = (prob >= 0.5).astype(_f32)
    lower_ref[...] = lower


def _heads(hb, nn, inh, wp1a, wp1b, bp1, wp2, bp2, wprob, bprob, wconf, bconf,
           wc1, bc1, wc2, bc2):
    args = (hb, nn, inh, wp1a, wp1b, bp1, wp2, bp2, wprob, bprob, wconf,
            bconf, wc1, bc1, wc2, bc2)
    return pl.pallas_call(
        _head_body,
        out_shape=[
            jax.ShapeDtypeStruct((B, 1), _f32),
            jax.ShapeDtypeStruct((B, 1), _f32),
            jax.ShapeDtypeStruct((B, 1), _f32),
            jax.ShapeDtypeStruct((B, 1), _f32),
            jax.ShapeDtypeStruct((B, H), _f32),
        ],
    )(*args)


def kernel(node_features, edge_index, edge_type, num_nodes,
           inheritance_features, W_in, b_in, W_msg, b_msg, Wz, Uz, bz,
           Wr, Ur, br, Wn, Un, bn, Wp1, bp1, Wp2, bp2, Wprob, bprob,
           Wconf, bconf, Wc1, bc1, Wc2, bc2):
    nf = node_features.reshape(N, F)
    h = _hin(nf, W_in, b_in.reshape(1, H))

    src = edge_index[0]
    dst = edge_index[1]
    pad = EPAD - E
    gidx = (edge_type * N + src).astype(jnp.int32)
    gidx_p = jnp.concatenate(
        [gidx, jnp.zeros((pad,), jnp.int32)]).reshape(EPAD // GLEN, GLEN)
    dst_p = jnp.concatenate(
        [dst.astype(jnp.int32),
         jnp.full((pad,), N, jnp.int32)]).reshape(EPAD // GLEN, GLEN)

    for l in range(L):
        table = _expand(h, W_msg[l].reshape(T * H, H),
                        b_msg[l].reshape(T, 1, H))
        parts = _edge_agg(table, gidx_p, dst_p)
        h = _gru([parts[i, :N] for i in range(NC)], h,
                 Wz[l], Uz[l], bz[l].reshape(1, H),
                 Wr[l], Ur[l], br[l].reshape(1, H),
                 Wn[l], Un[l], bn[l].reshape(1, H))

    logits, prob, conf, upper, lower = _heads(
        h.reshape(B, MAXN, H), num_nodes.reshape(B, 1),
        inheritance_features,
        Wp1[:8], Wp1[8:], bp1.reshape(1, PHID),
        Wp2, bp2.reshape(1, PHID),
        Wprob, bprob.reshape(1, 1),
        Wconf, bconf.reshape(1, 1),
        Wc1, bc1.reshape(1, H // 2),
        Wc2, bc2.reshape(1, 1))
    return logits, prob, conf, upper, lower
